# baseline (speedup 1.0000x reference)
import functools

import jax
import jax.numpy as jnp
from jax import lax
from jax.experimental import pallas as pl
from jax.experimental.pallas import tpu as pltpu
from jax.experimental.pallas import tpu_sc as plsc

_VOCAB = 1000
_VPAD = 1024
_B, _T = 1024, 50
_TPAD = 56
_NW = 32
_BPW = _B // _NW


def _sc_gather(idx_p, table_p):
    mesh = plsc.VectorSubcoreMesh(core_axis_name="c", subcore_axis_name="s")

    @functools.partial(
        pl.kernel,
        mesh=mesh,
        out_type=jax.ShapeDtypeStruct((_B, _TPAD, _VPAD), jnp.float32),
        scratch_types=[
            pltpu.VMEM((8, _TPAD), jnp.int32),
            pltpu.VMEM((8, _TPAD), jnp.int32),
            pltpu.VMEM((_TPAD, _VPAD), jnp.float32),
            pltpu.VMEM((_TPAD, _VPAD), jnp.float32),
            pltpu.SemaphoreType.DMA,
            pltpu.SemaphoreType.DMA,
            pltpu.SemaphoreType.DMA,
            pltpu.SemaphoreType.DMA,
            pltpu.SemaphoreType.DMA,
            pltpu.SemaphoreType.DMA,
        ],
        compiler_params=pltpu.CompilerParams(use_tc_tiling_on_sc=True),
    )
    def k(idx_hbm, table_hbm, out_hbm, i0, i1, buf0, buf1,
          gi0, gi1, g0, g1, s0, s1):
        ibufs = (i0, i1)
        bufs = (buf0, buf1)
        isems = (gi0, gi1)
        gsems = (g0, g1)
        ssems = (s0, s1)
        wid = lax.axis_index("s") * 2 + lax.axis_index("c")
        base = wid * _BPW

        for b in range(2):
            pltpu.sync_copy(idx_hbm.at[base + b], ibufs[b])
            pltpu.async_copy(table_hbm.at[ibufs[b].at[0]], bufs[b], gsems[b])

        @pl.loop(0, _BPW - 2, step=2)
        def _(j):
            for b in range(2):
                c = j + b
                pltpu.make_async_copy(
                    table_hbm.at[ibufs[b].at[0]], bufs[b], gsems[b]).wait()
                pltpu.async_copy(bufs[b], out_hbm.at[base + c], ssems[b])
                pltpu.async_copy(idx_hbm.at[base + c + 2], ibufs[b], isems[b])
                pltpu.make_async_copy(
                    bufs[b], out_hbm.at[base + c], ssems[b]).wait()
                pltpu.make_async_copy(
                    idx_hbm.at[base + c + 2], ibufs[b], isems[b]).wait()
                pltpu.async_copy(
                    table_hbm.at[ibufs[b].at[0]], bufs[b], gsems[b])

        for b in range(2):
            c = _BPW - 2 + b
            pltpu.make_async_copy(
                table_hbm.at[ibufs[b].at[0]], bufs[b], gsems[b]).wait()
            pltpu.sync_copy(bufs[b], out_hbm.at[base + c])

    return k(idx_p, table_p)


def kernel(idx, table):
    idx_p = jnp.pad(idx.reshape(_B, 1, _T).astype(jnp.int32),
                    ((0, 0), (0, 7), (0, _TPAD - _T)))
    table_p = jnp.pad(table, ((0, 0), (0, _VPAD - _VOCAB)))
    out = _sc_gather(idx_p, table_p)
    return out[:, :_T, :_VOCAB]


# speedup vs baseline: 2.2998x; 2.2998x over previous
import functools

import jax
import jax.numpy as jnp
from jax import lax
from jax.experimental import pallas as pl
from jax.experimental.pallas import tpu as pltpu
from jax.experimental.pallas import tpu_sc as plsc

_VOCAB = 1000
_VPAD = 1024
_B, _T = 1024, 50
_TPAD = 56
_NW = 32
_BPW = _B // _NW


def _sc_gather(idx_p, table_p):
    mesh = plsc.VectorSubcoreMesh(core_axis_name="c", subcore_axis_name="s")

    @functools.partial(
        pl.kernel,
        mesh=mesh,
        out_type=jax.ShapeDtypeStruct((_B, _TPAD, _VPAD), jnp.float32),
        scratch_types=[
            pltpu.VMEM((8, _TPAD), jnp.int32),
            pltpu.VMEM((8, _TPAD), jnp.int32),
            pltpu.VMEM((_TPAD, _VPAD), jnp.float32),
            pltpu.VMEM((_TPAD, _VPAD), jnp.float32),
            pltpu.SemaphoreType.DMA,
            pltpu.SemaphoreType.DMA,
            pltpu.SemaphoreType.DMA,
            pltpu.SemaphoreType.DMA,
            pltpu.SemaphoreType.DMA,
            pltpu.SemaphoreType.DMA,
        ],
        compiler_params=pltpu.CompilerParams(use_tc_tiling_on_sc=True),
    )
    def k(idx_hbm, table_hbm, out_hbm, i0, i1, buf0, buf1,
          gi0, gi1, g0, g1, s0, s1):
        ibufs = (i0, i1)
        bufs = (buf0, buf1)
        isems = (gi0, gi1)
        gsems = (g0, g1)
        ssems = (s0, s1)
        wid = lax.axis_index("s") * 2 + lax.axis_index("c")
        base = wid * _BPW

        for b in range(2):
            pltpu.sync_copy(idx_hbm.at[base + b], ibufs[b])
            pltpu.async_copy(table_hbm.at[ibufs[b].at[0]], bufs[b], gsems[b])

        @pl.loop(0, _BPW - 2, step=2)
        def _(j):
            for b in range(2):
                c = j + b
                pltpu.make_async_copy(
                    table_hbm.at[ibufs[b].at[0]], bufs[b], gsems[b]).wait()
                pltpu.async_copy(bufs[b], out_hbm.at[base + c], ssems[b])
                pltpu.async_copy(idx_hbm.at[base + c + 2], ibufs[b], isems[b])
                pltpu.make_async_copy(
                    bufs[b], out_hbm.at[base + c], ssems[b]).wait()
                pltpu.make_async_copy(
                    idx_hbm.at[base + c + 2], ibufs[b], isems[b]).wait()
                pltpu.async_copy(
                    table_hbm.at[ibufs[b].at[0]], bufs[b], gsems[b])

        for b in range(2):
            c = _BPW - 2 + b
            pltpu.make_async_copy(
                table_hbm.at[ibufs[b].at[0]], bufs[b], gsems[b]).wait()
            pltpu.sync_copy(bufs[b], out_hbm.at[base + c])

    return k(idx_p, table_p)


def kernel(idx, table):
    dummies = (jnp.arange(_B, dtype=jnp.int32)[:, None] * 7
               + jnp.arange(_TPAD - _T, dtype=jnp.int32)[None, :]) % _VOCAB
    idx_w = jnp.concatenate([idx.astype(jnp.int32), dummies], axis=1)
    idx_p = jnp.pad(idx_w.reshape(_B, 1, _TPAD), ((0, 0), (0, 7), (0, 0)))
    table_p = jnp.pad(table, ((0, 0), (0, _VPAD - _VOCAB)))
    out = _sc_gather(idx_p, table_p)
    return out[:, :_T, :_VOCAB]
